# baseline (speedup 1.0000x reference)
import functools

import jax
import jax.numpy as jnp
from jax import lax
from jax.experimental import pallas as pl
from jax.experimental.pallas import tpu as pltpu
from jax.experimental.pallas import tpu_sc as plsc

_B = 16
_N = 20000
_C = 25
_NCLS = 21
_HALF = _N // 2
_CH = 2000
_NCHUNK = _HALF // _CH
_GROUPS = _CH // 16


def _decode_body(logits_hbm, db_hbm, boxes_hbm, scores_hbm, classes_hbm,
                 counts_hbm, lbuf, dbuf, bbuf, sbuf, cbuf, cntbuf):
    nc = 2
    wid = lax.axis_index("s") * nc + lax.axis_index("c")
    b = wid // 2
    half = wid % 2
    row0 = b * _N + half * _HALF

    iota = lax.iota(jnp.int32, 16)
    zeros_i = jnp.zeros((16,), jnp.int32)

    cnt = zeros_i
    for chunk in range(_NCHUNK):
        base = row0 + chunk * _CH
        dbase = half * _HALF + chunk * _CH
        pltpu.sync_copy(logits_hbm.at[pl.ds(base * _C, _CH * _C)], lbuf)
        pltpu.sync_copy(db_hbm.at[pl.ds(dbase * 4, _CH * 4)], dbuf)

        def group(g, cnt):
            rows = g * 16 + iota
            lb = rows * _C
            l0 = plsc.load_gather(lbuf, [lb])
            l1 = plsc.load_gather(lbuf, [lb + 1])
            l2 = plsc.load_gather(lbuf, [lb + 2])
            l3 = plsc.load_gather(lbuf, [lb + 3])
            m = plsc.load_gather(lbuf, [lb + 4])
            cls = zeros_i
            for c in range(1, _NCLS):
                v = plsc.load_gather(lbuf, [lb + 4 + c])
                gt = v > m
                m = jnp.where(gt, v, m)
                cls = jnp.where(gt, c, cls)
            db4 = rows * 4
            x0 = plsc.load_gather(dbuf, [db4])
            y0 = plsc.load_gather(dbuf, [db4 + 1])
            x1 = plsc.load_gather(dbuf, [db4 + 2])
            y1 = plsc.load_gather(dbuf, [db4 + 3])
            cx = (x1 + x0) * 0.5
            cy = (y1 + y0) * 0.5
            w = x1 - x0
            h = y1 - y0
            p0 = l0 * w + cx
            p1 = l1 * h + cy
            bw = jnp.exp(l2) * w
            bh = jnp.exp(l3) * h
            mask = cls != 0
            fm = mask.astype(jnp.float32)
            bx0 = jnp.clip(p0 - bw * 0.5, 0.0, 1.0) * fm
            by0 = jnp.clip(p1 - bh * 0.5, 0.0, 1.0) * fm
            bx1 = jnp.clip(p0 + bw * 0.5, 0.0, 1.0) * fm
            by1 = jnp.clip(p1 + bh * 0.5, 0.0, 1.0) * fm
            plsc.store_scatter(bbuf, [db4], bx0)
            plsc.store_scatter(bbuf, [db4 + 1], by0)
            plsc.store_scatter(bbuf, [db4 + 2], bx1)
            plsc.store_scatter(bbuf, [db4 + 3], by1)
            sbuf[pl.ds(g * 16, 16)] = m * fm
            cbuf[pl.ds(g * 16, 16)] = cls
            return cnt + mask.astype(jnp.int32)

        cnt = lax.fori_loop(0, _GROUPS, group, cnt)
        pltpu.sync_copy(bbuf, boxes_hbm.at[pl.ds(base * 4, _CH * 4)])
        pltpu.sync_copy(sbuf, scores_hbm.at[pl.ds(base, _CH)])
        pltpu.sync_copy(cbuf, classes_hbm.at[pl.ds(base, _CH)])

    cntbuf[...] = cnt
    pltpu.sync_copy(cntbuf, counts_hbm.at[wid])


@functools.partial(jax.jit, static_argnames=())
def _run(logits_flat, db_flat):
    mesh = plsc.VectorSubcoreMesh(core_axis_name="c", subcore_axis_name="s")
    kern = functools.partial(
        pl.kernel,
        mesh=mesh,
        out_type=[
            jax.ShapeDtypeStruct((_B * _N * 4,), jnp.float32),
            jax.ShapeDtypeStruct((_B * _N,), jnp.float32),
            jax.ShapeDtypeStruct((_B * _N,), jnp.int32),
            jax.ShapeDtypeStruct((32, 16), jnp.int32),
        ],
        scratch_types=[
            pltpu.VMEM((_CH * _C,), jnp.float32),
            pltpu.VMEM((_CH * 4,), jnp.float32),
            pltpu.VMEM((_CH * 4,), jnp.float32),
            pltpu.VMEM((_CH,), jnp.float32),
            pltpu.VMEM((_CH,), jnp.int32),
            pltpu.VMEM((16,), jnp.int32),
        ],
        compiler_params=pltpu.CompilerParams(needs_layout_passes=False),
    )(_decode_body)
    return kern(logits_flat, db_flat)


def kernel(logits, default_boxes):
    boxes, scores, classes, counts = _run(
        logits.reshape(-1), default_boxes.reshape(-1))
    detection_boxes = boxes.reshape(_B, _N, 4)
    detection_scores = scores.reshape(_B, _N)
    detection_classes = classes.reshape(_B, _N)
    detection_num = counts.reshape(_B, 32).sum(axis=-1).astype(jnp.int32)
    return detection_boxes, detection_scores, detection_classes, detection_num


# speedup vs baseline: 7.3600x; 7.3600x over previous
import functools

import jax
import jax.numpy as jnp
from jax import lax
from jax.experimental import pallas as pl
from jax.experimental.pallas import tpu as pltpu
from jax.experimental.pallas import tpu_sc as plsc

_B = 16
_N = 20000
_C = 25
_NCLS = 21
_CH = 768
_MAIN = 9984
_NCHUNK = _MAIN // _CH
_TAIL = _N - 2 * _MAIN
_GROUPS = _CH // 16


def _process(lbuf, dbuf, bbuf, sbuf, cbuf, j):
    sl = pl.ds(j * 16, 16)
    l0 = lbuf[0, 0, sl]
    l1 = lbuf[1, 0, sl]
    l2 = lbuf[2, 0, sl]
    l3 = lbuf[3, 0, sl]
    vals = [lbuf[4 + c, 0, sl] for c in range(_NCLS)]
    idxs = [jnp.full((16,), c, jnp.int32) for c in range(_NCLS)]
    while len(vals) > 1:
        nv, ni = [], []
        for k in range(0, len(vals) - 1, 2):
            ge = vals[k] >= vals[k + 1]
            nv.append(jnp.maximum(vals[k], vals[k + 1]))
            ni.append(jnp.where(ge, idxs[k], idxs[k + 1]))
        if len(vals) % 2:
            nv.append(vals[-1])
            ni.append(idxs[-1])
        vals, idxs = nv, ni
    m, cls = vals[0], idxs[0]

    x0 = dbuf[0, sl]
    y0 = dbuf[1, sl]
    x1 = dbuf[2, sl]
    y1 = dbuf[3, sl]
    cx = (x1 + x0) * 0.5
    cy = (y1 + y0) * 0.5
    w = x1 - x0
    h = y1 - y0
    p0 = l0 * w + cx
    p1 = l1 * h + cy
    bw = jnp.exp(l2) * w
    bh = jnp.exp(l3) * h
    mask = cls != 0
    fm = mask.astype(jnp.float32)
    bbuf[0, 0, sl] = jnp.clip(p0 - bw * 0.5, 0.0, 1.0) * fm
    bbuf[0, 1, sl] = jnp.clip(p1 - bh * 0.5, 0.0, 1.0) * fm
    bbuf[0, 2, sl] = jnp.clip(p0 + bw * 0.5, 0.0, 1.0) * fm
    bbuf[0, 3, sl] = jnp.clip(p1 + bh * 0.5, 0.0, 1.0) * fm
    sbuf[0, sl] = m * fm
    cbuf[0, sl] = cls
    return mask.astype(jnp.int32)


def _decode_body(lt, dbt, bx, sc, cl, cnts,
                 lbuf, dbuf, bbuf, sbuf, cbuf,
                 tlbuf, tdbuf, tbbuf, tsbuf, tcbuf, cntbuf):
    nc = 2
    wid = lax.axis_index("s") * nc + lax.axis_index("c")
    b = wid // 2
    half = wid % 2

    cnt = jnp.zeros((16,), jnp.int32)
    for chunk in range(_NCHUNK):
        n0 = half * _MAIN + chunk * _CH
        nsl = pl.ds(n0, _CH)
        pltpu.sync_copy(lt.at[pl.ds(0, _C), pl.ds(b, 1), nsl], lbuf)
        pltpu.sync_copy(dbt.at[pl.ds(0, 4), nsl], dbuf)

        def group(j, cnt):
            return cnt + _process(lbuf, dbuf, bbuf, sbuf, cbuf, j)

        cnt = lax.fori_loop(0, _GROUPS, group, cnt)
        pltpu.sync_copy(bbuf, bx.at[pl.ds(b, 1), pl.ds(0, 4), nsl])
        pltpu.sync_copy(sbuf, sc.at[pl.ds(b, 1), nsl])
        pltpu.sync_copy(cbuf, cl.at[pl.ds(b, 1), nsl])

    tn = pl.ds(2 * _MAIN, _TAIL)
    pltpu.sync_copy(lt.at[pl.ds(0, _C), pl.ds(b, 1), tn], tlbuf)
    pltpu.sync_copy(dbt.at[pl.ds(0, 4), tn], tdbuf)
    tinc = jnp.zeros((16,), jnp.int32)
    for j in range(_TAIL // 16):
        tinc = tinc + _process(tlbuf, tdbuf, tbbuf, tsbuf, tcbuf, j)
    cnt = cnt + jnp.where(half == 1, tinc, 0)
    pltpu.sync_copy(tbbuf, bx.at[pl.ds(b, 1), pl.ds(0, 4), tn])
    pltpu.sync_copy(tsbuf, sc.at[pl.ds(b, 1), tn])
    pltpu.sync_copy(tcbuf, cl.at[pl.ds(b, 1), tn])

    cntbuf[...] = cnt
    pltpu.sync_copy(cntbuf, cnts.at[wid])


@jax.jit
def _run(lt, dbt):
    mesh = plsc.VectorSubcoreMesh(core_axis_name="c", subcore_axis_name="s")
    kern = functools.partial(
        pl.kernel,
        mesh=mesh,
        out_type=[
            jax.ShapeDtypeStruct((_B, 4, _N), jnp.float32),
            jax.ShapeDtypeStruct((_B, _N), jnp.float32),
            jax.ShapeDtypeStruct((_B, _N), jnp.int32),
            jax.ShapeDtypeStruct((32, 16), jnp.int32),
        ],
        scratch_types=[
            pltpu.VMEM((_C, 1, _CH), jnp.float32),
            pltpu.VMEM((4, _CH), jnp.float32),
            pltpu.VMEM((1, 4, _CH), jnp.float32),
            pltpu.VMEM((1, _CH), jnp.float32),
            pltpu.VMEM((1, _CH), jnp.int32),
            pltpu.VMEM((_C, 1, _TAIL), jnp.float32),
            pltpu.VMEM((4, _TAIL), jnp.float32),
            pltpu.VMEM((1, 4, _TAIL), jnp.float32),
            pltpu.VMEM((1, _TAIL), jnp.float32),
            pltpu.VMEM((1, _TAIL), jnp.int32),
            pltpu.VMEM((16,), jnp.int32),
        ],
        compiler_params=pltpu.CompilerParams(
            needs_layout_passes=False, use_tc_tiling_on_sc=True),
    )(_decode_body)
    return kern(lt, dbt)


def kernel(logits, default_boxes):
    lt = jnp.transpose(logits, (2, 0, 1))
    dbt = jnp.transpose(default_boxes, (1, 0))
    boxes, scores, classes, counts = _run(lt, dbt)
    detection_boxes = jnp.transpose(boxes, (0, 2, 1))
    detection_num = counts.reshape(_B, 32).sum(axis=-1).astype(jnp.int32)
    return detection_boxes, scores, classes, detection_num


# speedup vs baseline: 10.7765x; 1.4642x over previous
import functools

import jax
import jax.numpy as jnp
from jax import lax
from jax.experimental import pallas as pl
from jax.experimental.pallas import tpu as pltpu
from jax.experimental.pallas import tpu_sc as plsc

_B = 16
_N = 20000
_C = 25
_NCLS = 21
_CH = 768
_MAIN = 9984
_NCHUNK = _MAIN // _CH
_TAIL = _N - 2 * _MAIN
_GROUPS = _CH // 16


def _process(lbuf, dbuf, bbuf, sbuf, cbuf, j):
    sl = pl.ds(j * 16, 16)
    l0 = lbuf[0, 0, sl]
    l1 = lbuf[1, 0, sl]
    l2 = lbuf[2, 0, sl]
    l3 = lbuf[3, 0, sl]
    vals = [lbuf[4 + c, 0, sl] for c in range(_NCLS)]
    idxs = [jnp.full((16,), c, jnp.int32) for c in range(_NCLS)]
    while len(vals) > 1:
        nv, ni = [], []
        for k in range(0, len(vals) - 1, 2):
            ge = vals[k] >= vals[k + 1]
            nv.append(jnp.maximum(vals[k], vals[k + 1]))
            ni.append(jnp.where(ge, idxs[k], idxs[k + 1]))
        if len(vals) % 2:
            nv.append(vals[-1])
            ni.append(idxs[-1])
        vals, idxs = nv, ni
    m, cls = vals[0], idxs[0]

    x0 = dbuf[0, sl]
    y0 = dbuf[1, sl]
    x1 = dbuf[2, sl]
    y1 = dbuf[3, sl]
    cx = (x1 + x0) * 0.5
    cy = (y1 + y0) * 0.5
    w = x1 - x0
    h = y1 - y0
    p0 = l0 * w + cx
    p1 = l1 * h + cy
    bw = jnp.exp(l2) * w
    bh = jnp.exp(l3) * h
    mask = cls != 0
    fm = mask.astype(jnp.float32)
    bbuf[0, 0, sl] = jnp.clip(p0 - bw * 0.5, 0.0, 1.0) * fm
    bbuf[0, 1, sl] = jnp.clip(p1 - bh * 0.5, 0.0, 1.0) * fm
    bbuf[0, 2, sl] = jnp.clip(p0 + bw * 0.5, 0.0, 1.0) * fm
    bbuf[0, 3, sl] = jnp.clip(p1 + bh * 0.5, 0.0, 1.0) * fm
    sbuf[0, sl] = m * fm
    cbuf[0, sl] = cls
    return mask.astype(jnp.int32)


def _decode_body(lt, dbt, bx, sc, cl, cnts,
                 lbuf0, lbuf1, dbuf0, dbuf1, bbuf0, bbuf1,
                 sbuf0, sbuf1, cbuf0, cbuf1,
                 tlbuf, tdbuf, tbbuf, tsbuf, tcbuf, cntbuf,
                 sin0, sin1, sout0, sout1):
    b = lax.axis_index("s")
    half = lax.axis_index("c")
    wid = b * 2 + half

    lbufs = (lbuf0, lbuf1)
    dbufs = (dbuf0, dbuf1)
    bbufs = (bbuf0, bbuf1)
    sbufs = (sbuf0, sbuf1)
    cbufs = (cbuf0, cbuf1)
    sins = (sin0, sin1)
    souts = (sout0, sout1)

    def nsl(chunk):
        return pl.ds(half * _MAIN + chunk * _CH, _CH)

    def start_in(chunk, slot):
        c1 = pltpu.make_async_copy(
            lt.at[pl.ds(0, _C), pl.ds(b, 1), nsl(chunk)], lbufs[slot],
            sins[slot])
        c1.start()
        c2 = pltpu.make_async_copy(
            dbt.at[pl.ds(0, 4), nsl(chunk)], dbufs[slot], sins[slot])
        c2.start()
        return (c1, c2)

    def start_out(chunk, slot):
        c1 = pltpu.make_async_copy(
            bbufs[slot], bx.at[pl.ds(b, 1), pl.ds(0, 4), nsl(chunk)],
            souts[slot])
        c1.start()
        c2 = pltpu.make_async_copy(
            sbufs[slot], sc.at[pl.ds(b, 1), nsl(chunk)], souts[slot])
        c2.start()
        c3 = pltpu.make_async_copy(
            cbufs[slot], cl.at[pl.ds(b, 1), nsl(chunk)], souts[slot])
        c3.start()
        return (c1, c2, c3)

    cnt = jnp.zeros((16,), jnp.int32)
    pend_in = [None, None]
    pend_out = [None, None]
    pend_in[0] = start_in(0, 0)
    for chunk in range(_NCHUNK):
        slot = chunk % 2
        if chunk + 1 < _NCHUNK:
            pend_in[1 - slot] = start_in(chunk + 1, 1 - slot)
        for h in pend_in[slot]:
            h.wait()
        if pend_out[slot] is not None:
            for h in pend_out[slot]:
                h.wait()
            pend_out[slot] = None

        def group(j, cnt, slot=slot):
            return cnt + _process(lbufs[slot], dbufs[slot], bbufs[slot],
                                  sbufs[slot], cbufs[slot], j)

        cnt = lax.fori_loop(0, _GROUPS, group, cnt)
        pend_out[slot] = start_out(chunk, slot)

    for s in (0, 1):
        if pend_out[s] is not None:
            for h in pend_out[s]:
                h.wait()

    tn = pl.ds(2 * _MAIN, _TAIL)
    pltpu.sync_copy(lt.at[pl.ds(0, _C), pl.ds(b, 1), tn], tlbuf)
    pltpu.sync_copy(dbt.at[pl.ds(0, 4), tn], tdbuf)
    tinc = jnp.zeros((16,), jnp.int32)
    for j in range(_TAIL // 16):
        tinc = tinc + _process(tlbuf, tdbuf, tbbuf, tsbuf, tcbuf, j)
    cnt = cnt + jnp.where(half == 1, tinc, 0)
    pltpu.sync_copy(tbbuf, bx.at[pl.ds(b, 1), pl.ds(0, 4), tn])
    pltpu.sync_copy(tsbuf, sc.at[pl.ds(b, 1), tn])
    pltpu.sync_copy(tcbuf, cl.at[pl.ds(b, 1), tn])

    cntbuf[...] = cnt
    pltpu.sync_copy(cntbuf, cnts.at[wid])


@jax.jit
def _run(lt, dbt):
    mesh = plsc.VectorSubcoreMesh(core_axis_name="c", subcore_axis_name="s")
    kern = functools.partial(
        pl.kernel,
        mesh=mesh,
        out_type=[
            jax.ShapeDtypeStruct((_B, 4, _N), jnp.float32),
            jax.ShapeDtypeStruct((_B, _N), jnp.float32),
            jax.ShapeDtypeStruct((_B, _N), jnp.int32),
            jax.ShapeDtypeStruct((32, 16), jnp.int32),
        ],
        scratch_types=[
            pltpu.VMEM((_C, 1, _CH), jnp.float32),
            pltpu.VMEM((_C, 1, _CH), jnp.float32),
            pltpu.VMEM((4, _CH), jnp.float32),
            pltpu.VMEM((4, _CH), jnp.float32),
            pltpu.VMEM((1, 4, _CH), jnp.float32),
            pltpu.VMEM((1, 4, _CH), jnp.float32),
            pltpu.VMEM((1, _CH), jnp.float32),
            pltpu.VMEM((1, _CH), jnp.float32),
            pltpu.VMEM((1, _CH), jnp.int32),
            pltpu.VMEM((1, _CH), jnp.int32),
            pltpu.VMEM((_C, 1, _TAIL), jnp.float32),
            pltpu.VMEM((4, _TAIL), jnp.float32),
            pltpu.VMEM((1, 4, _TAIL), jnp.float32),
            pltpu.VMEM((1, _TAIL), jnp.float32),
            pltpu.VMEM((1, _TAIL), jnp.int32),
            pltpu.VMEM((16,), jnp.int32),
            pltpu.SemaphoreType.DMA,
            pltpu.SemaphoreType.DMA,
            pltpu.SemaphoreType.DMA,
            pltpu.SemaphoreType.DMA,
        ],
        compiler_params=pltpu.CompilerParams(
            needs_layout_passes=False, use_tc_tiling_on_sc=True),
    )(_decode_body)
    return kern(lt, dbt)


def kernel(logits, default_boxes):
    lt = jnp.transpose(logits, (2, 0, 1))
    dbt = jnp.transpose(default_boxes, (1, 0))
    boxes, scores, classes, counts = _run(lt, dbt)
    detection_boxes = jnp.transpose(boxes, (0, 2, 1))
    detection_num = counts.reshape(_B, 32).sum(axis=-1).astype(jnp.int32)
    return detection_boxes, scores, classes, detection_num


# speedup vs baseline: 11.0302x; 1.0235x over previous
import functools

import jax
import jax.numpy as jnp
from jax import lax
from jax.experimental import pallas as pl
from jax.experimental.pallas import tpu as pltpu
from jax.experimental.pallas import tpu_sc as plsc

_B = 16
_N = 20000
_C = 25
_NCLS = 21
_COLS = 156
_TAIL0 = _COLS * 128
_TAIL = _N - _TAIL0
_NCHUNK = 10


def _argmax_tree(vals):
    idxs = [jnp.full((16,), c, jnp.int32) for c in range(len(vals))]
    while len(vals) > 1:
        nv, ni = [], []
        for k in range(0, len(vals) - 1, 2):
            ge = vals[k] >= vals[k + 1]
            nv.append(jnp.maximum(vals[k], vals[k + 1]))
            ni.append(jnp.where(ge, idxs[k], idxs[k + 1]))
        if len(vals) % 2:
            nv.append(vals[-1])
            ni.append(idxs[-1])
        vals, idxs = nv, ni
    return vals[0], idxs[0]


def _process(lbuf, dbuf, bbuf, sbuf, cbuf, r, sl):
    l0 = lbuf[0, r, sl]
    l1 = lbuf[1, r, sl]
    l2 = lbuf[2, r, sl]
    l3 = lbuf[3, r, sl]
    m, cls = _argmax_tree([lbuf[4 + c, r, sl] for c in range(_NCLS)])

    x0 = dbuf[0, sl]
    y0 = dbuf[1, sl]
    x1 = dbuf[2, sl]
    y1 = dbuf[3, sl]
    cx = (x1 + x0) * 0.5
    cy = (y1 + y0) * 0.5
    w = x1 - x0
    h = y1 - y0
    p0 = l0 * w + cx
    p1 = l1 * h + cy
    bw = jnp.exp(l2) * w
    bh = jnp.exp(l3) * h
    mask = cls != 0
    fm = mask.astype(jnp.float32)
    bbuf[r, 0, sl] = jnp.clip(p0 - bw * 0.5, 0.0, 1.0) * fm
    bbuf[r, 1, sl] = jnp.clip(p1 - bh * 0.5, 0.0, 1.0) * fm
    bbuf[r, 2, sl] = jnp.clip(p0 + bw * 0.5, 0.0, 1.0) * fm
    bbuf[r, 3, sl] = jnp.clip(p1 + bh * 0.5, 0.0, 1.0) * fm
    sbuf[r, sl] = m * fm
    cbuf[r, sl] = cls
    return mask.astype(jnp.int32)


def _decode_body(lt, dbt, bx, sc, cl, cnts,
                 lbuf0, lbuf1, dbuf0, dbuf1, bbuf0, bbuf1,
                 sbuf0, sbuf1, cbuf0, cbuf1,
                 tlbuf, tdbuf, tbbuf, tsbuf, tcbuf, cntbuf, stbuf,
                 sin0, sin1, sout0, sout1):
    cg = lax.axis_index("s")
    tb = lax.axis_index("c")
    wid = cg * 2 + tb
    r0 = 8 * tb

    start = cg * 9 + jnp.minimum(cg, 12)

    lbufs = (lbuf0, lbuf1)
    dbufs = (dbuf0, dbuf1)
    bbufs = (bbuf0, bbuf1)
    sbufs = (sbuf0, sbuf1)
    cbufs = (cbuf0, cbuf1)
    sins = (sin0, sin1)
    souts = (sout0, sout1)

    iota = lax.iota(jnp.int32, 16)
    for r in range(16):
        cntbuf[pl.ds(16 * r, 16)] = jnp.zeros((16,), jnp.int32)

    def col_of(k):
        return jnp.minimum(start + k, _COLS - 1)

    def nsl(k):
        return pl.ds(col_of(k) * 128, 128)

    def start_in(k, slot):
        pltpu.make_async_copy(
            lt.at[pl.ds(0, _C), pl.ds(r0, 8), nsl(k)], lbufs[slot],
            sins[slot]).start()
        pltpu.make_async_copy(
            dbt.at[pl.ds(0, 4), nsl(k)], dbufs[slot], sins[slot]).start()

    def start_out(k, slot):
        pltpu.make_async_copy(
            bbufs[slot], bx.at[pl.ds(r0, 8), pl.ds(0, 4), nsl(k)],
            souts[slot]).start()
        pltpu.make_async_copy(
            sbufs[slot], sc.at[pl.ds(r0, 8), nsl(k)], souts[slot]).start()
        pltpu.make_async_copy(
            cbufs[slot], cl.at[pl.ds(r0, 8), nsl(k)], souts[slot]).start()

    def wait_in(slot):
        pltpu.make_async_copy(
            lt.at[pl.ds(0, _C), pl.ds(r0, 8), nsl(0)], lbufs[slot],
            sins[slot]).wait()
        pltpu.make_async_copy(
            dbt.at[pl.ds(0, 4), nsl(0)], dbufs[slot], sins[slot]).wait()

    def wait_out(slot):
        pltpu.make_async_copy(
            bbufs[slot], bx.at[pl.ds(r0, 8), pl.ds(0, 4), nsl(0)],
            souts[slot]).wait()
        pltpu.make_async_copy(
            sbufs[slot], sc.at[pl.ds(r0, 8), nsl(0)], souts[slot]).wait()
        pltpu.make_async_copy(
            cbufs[slot], cl.at[pl.ds(r0, 8), nsl(0)], souts[slot]).wait()

    def compute(k, slot):
        owned = jnp.logical_or(cg < 12, k < 9)

        def group(i, _):
            r = i // 8
            g = i - r * 8
            sl = pl.ds(g * 16, 16)
            inc = _process(lbufs[slot], dbufs[slot], bbufs[slot],
                           sbufs[slot], cbufs[slot], r, sl)
            inc = jnp.where(owned, inc, 0)
            csl = pl.ds(r * 16, 16)
            cntbuf[csl] = cntbuf[csl] + inc
            return 0

        lax.fori_loop(0, 64, group, 0)

    start_in(0, 0)
    start_in(1, 1)
    start_out(1, 1)
    wait_in(0)
    compute(0, 0)
    start_out(0, 0)

    def pair(k, carry):
        a = 2 * k + 1
        bb = 2 * k + 2
        start_in(bb, 0)
        wait_in(1)
        wait_out(1)
        compute(a, 1)
        start_out(a, 1)
        start_in(a + 2, 1)
        wait_in(0)
        wait_out(0)
        compute(bb, 0)
        start_out(bb, 0)
        return carry

    lax.fori_loop(0, (_NCHUNK - 2) // 2, pair, 0)
    wait_in(1)
    wait_out(1)
    compute(_NCHUNK - 1, 1)
    start_out(_NCHUNK - 1, 1)
    wait_out(0)
    wait_out(1)

    @pl.when(cg == 0)
    def _tail():
        tn = pl.ds(_TAIL0, _TAIL)
        pltpu.sync_copy(lt.at[pl.ds(0, _C), pl.ds(r0, 8), tn], tlbuf)
        pltpu.sync_copy(dbt.at[pl.ds(0, 4), tn], tdbuf)
        for r in range(8):
            for g in range(_TAIL // 16):
                inc = _process(tlbuf, tdbuf, tbbuf, tsbuf, tcbuf,
                               r, pl.ds(g * 16, 16))
                csl = pl.ds(r * 16, 16)
                cntbuf[csl] = cntbuf[csl] + inc
        pltpu.sync_copy(tbbuf, bx.at[pl.ds(r0, 8), pl.ds(0, 4), tn])
        pltpu.sync_copy(tsbuf, sc.at[pl.ds(r0, 8), tn])
        pltpu.sync_copy(tcbuf, cl.at[pl.ds(r0, 8), tn])

    rs = jnp.zeros((16,), jnp.int32)
    for j in range(16):
        rs = rs + plsc.load_gather(cntbuf, [iota * 16 + j])
    stbuf[...] = rs
    pltpu.sync_copy(stbuf, cnts.at[wid])


@jax.jit
def _run(lt, dbt):
    mesh = plsc.VectorSubcoreMesh(core_axis_name="c", subcore_axis_name="s")
    kern = functools.partial(
        pl.kernel,
        mesh=mesh,
        out_type=[
            jax.ShapeDtypeStruct((_B, 4, _N), jnp.float32),
            jax.ShapeDtypeStruct((_B, _N), jnp.float32),
            jax.ShapeDtypeStruct((_B, _N), jnp.int32),
            jax.ShapeDtypeStruct((32, 16), jnp.int32),
        ],
        scratch_types=[
            pltpu.VMEM((_C, 8, 128), jnp.float32),
            pltpu.VMEM((_C, 8, 128), jnp.float32),
            pltpu.VMEM((4, 128), jnp.float32),
            pltpu.VMEM((4, 128), jnp.float32),
            pltpu.VMEM((8, 4, 128), jnp.float32),
            pltpu.VMEM((8, 4, 128), jnp.float32),
            pltpu.VMEM((8, 128), jnp.float32),
            pltpu.VMEM((8, 128), jnp.float32),
            pltpu.VMEM((8, 128), jnp.int32),
            pltpu.VMEM((8, 128), jnp.int32),
            pltpu.VMEM((_C, 8, _TAIL), jnp.float32),
            pltpu.VMEM((4, _TAIL), jnp.float32),
            pltpu.VMEM((8, 4, _TAIL), jnp.float32),
            pltpu.VMEM((8, _TAIL), jnp.float32),
            pltpu.VMEM((8, _TAIL), jnp.int32),
            pltpu.VMEM((256,), jnp.int32),
            pltpu.VMEM((16,), jnp.int32),
            pltpu.SemaphoreType.DMA,
            pltpu.SemaphoreType.DMA,
            pltpu.SemaphoreType.DMA,
            pltpu.SemaphoreType.DMA,
        ],
        compiler_params=pltpu.CompilerParams(
            needs_layout_passes=False, use_tc_tiling_on_sc=True),
    )(_decode_body)
    return kern(lt, dbt)


def kernel(logits, default_boxes):
    lt = jnp.transpose(logits, (2, 0, 1))
    dbt = jnp.transpose(default_boxes, (1, 0))
    boxes, scores, classes, counts = _run(lt, dbt)
    detection_boxes = jnp.transpose(boxes, (0, 2, 1))
    sums = counts.reshape(_B, 2, 16).sum(axis=0)
    detection_num = sums[:, :8].reshape(_B).astype(jnp.int32)
    return detection_boxes, scores, classes, detection_num


# speedup vs baseline: 11.4512x; 1.0382x over previous
import functools

import jax
import jax.numpy as jnp
from jax import lax
from jax.experimental import pallas as pl
from jax.experimental.pallas import tpu as pltpu
from jax.experimental.pallas import tpu_sc as plsc

_B = 16
_N = 20000
_C = 25
_NCLS = 21
_CH = 1664
_MAIN = 9984
_NCHUNK = _MAIN // _CH
_TAIL = _N - 2 * _MAIN
_GROUPS = _CH // 16


def _process(lbuf, dbuf, bbuf, sbuf, cbuf, j):
    sl = pl.ds(j * 16, 16)
    l0 = lbuf[0, 0, sl]
    l1 = lbuf[1, 0, sl]
    l2 = lbuf[2, 0, sl]
    l3 = lbuf[3, 0, sl]
    vals = [lbuf[4 + c, 0, sl] for c in range(_NCLS)]
    idxs = [jnp.full((16,), c, jnp.int32) for c in range(_NCLS)]
    while len(vals) > 1:
        nv, ni = [], []
        for k in range(0, len(vals) - 1, 2):
            ge = vals[k] >= vals[k + 1]
            nv.append(jnp.maximum(vals[k], vals[k + 1]))
            ni.append(jnp.where(ge, idxs[k], idxs[k + 1]))
        if len(vals) % 2:
            nv.append(vals[-1])
            ni.append(idxs[-1])
        vals, idxs = nv, ni
    m, cls = vals[0], idxs[0]

    x0 = dbuf[0, sl]
    y0 = dbuf[1, sl]
    x1 = dbuf[2, sl]
    y1 = dbuf[3, sl]
    cx = (x1 + x0) * 0.5
    cy = (y1 + y0) * 0.5
    w = x1 - x0
    h = y1 - y0
    p0 = l0 * w + cx
    p1 = l1 * h + cy
    bw = jnp.exp(l2) * w
    bh = jnp.exp(l3) * h
    mask = cls != 0
    fm = mask.astype(jnp.float32)
    bbuf[0, 0, sl] = jnp.clip(p0 - bw * 0.5, 0.0, 1.0) * fm
    bbuf[0, 1, sl] = jnp.clip(p1 - bh * 0.5, 0.0, 1.0) * fm
    bbuf[0, 2, sl] = jnp.clip(p0 + bw * 0.5, 0.0, 1.0) * fm
    bbuf[0, 3, sl] = jnp.clip(p1 + bh * 0.5, 0.0, 1.0) * fm
    sbuf[0, sl] = m * fm
    cbuf[0, sl] = cls
    return mask.astype(jnp.int32)


def _decode_body(lt, dbt, bx, sc, cl, cnts,
                 lbuf0, lbuf1, dbuf0, dbuf1, bbuf0, bbuf1,
                 sbuf0, sbuf1, cbuf0, cbuf1,
                 tlbuf, tdbuf, tbbuf, tsbuf, tcbuf, cntbuf,
                 sin0, sin1, sout0, sout1):
    b = lax.axis_index("s")
    half = lax.axis_index("c")
    wid = b * 2 + half

    lbufs = (lbuf0, lbuf1)
    dbufs = (dbuf0, dbuf1)
    bbufs = (bbuf0, bbuf1)
    sbufs = (sbuf0, sbuf1)
    cbufs = (cbuf0, cbuf1)
    sins = (sin0, sin1)
    souts = (sout0, sout1)

    def nsl(chunk):
        return pl.ds(half * _MAIN + chunk * _CH, _CH)

    def start_in(chunk, slot):
        c1 = pltpu.make_async_copy(
            lt.at[pl.ds(0, _C), pl.ds(b, 1), nsl(chunk)], lbufs[slot],
            sins[slot])
        c1.start()
        c2 = pltpu.make_async_copy(
            dbt.at[pl.ds(0, 4), nsl(chunk)], dbufs[slot], sins[slot])
        c2.start()
        return (c1, c2)

    def start_out(chunk, slot):
        c1 = pltpu.make_async_copy(
            bbufs[slot], bx.at[pl.ds(b, 1), pl.ds(0, 4), nsl(chunk)],
            souts[slot])
        c1.start()
        c2 = pltpu.make_async_copy(
            sbufs[slot], sc.at[pl.ds(b, 1), nsl(chunk)], souts[slot])
        c2.start()
        c3 = pltpu.make_async_copy(
            cbufs[slot], cl.at[pl.ds(b, 1), nsl(chunk)], souts[slot])
        c3.start()
        return (c1, c2, c3)

    def wait_in(slot):
        pltpu.make_async_copy(
            lt.at[pl.ds(0, _C), pl.ds(b, 1), nsl(0)], lbufs[slot],
            sins[slot]).wait()
        pltpu.make_async_copy(
            dbt.at[pl.ds(0, 4), nsl(0)], dbufs[slot], sins[slot]).wait()

    def wait_out(slot):
        pltpu.make_async_copy(
            bbufs[slot], bx.at[pl.ds(b, 1), pl.ds(0, 4), nsl(0)],
            souts[slot]).wait()
        pltpu.make_async_copy(
            sbufs[slot], sc.at[pl.ds(b, 1), nsl(0)], souts[slot]).wait()
        pltpu.make_async_copy(
            cbufs[slot], cl.at[pl.ds(b, 1), nsl(0)], souts[slot]).wait()

    def compute(slot, cnt):
        def group(j, cnt):
            return cnt + _process(lbufs[slot], dbufs[slot], bbufs[slot],
                                  sbufs[slot], cbufs[slot], j)
        return lax.fori_loop(0, _GROUPS, group, cnt)

    assert _NCHUNK % 2 == 0 and _NCHUNK >= 4
    cnt = jnp.zeros((16,), jnp.int32)
    start_in(0, 0)
    start_in(1, 1)
    start_out(1, 1)
    wait_in(0)
    cnt = compute(0, cnt)
    start_out(0, 0)

    def pair(k, cnt):
        a = 2 * k + 1
        bb = 2 * k + 2
        start_in(bb, 0)
        wait_in(1)
        wait_out(1)
        cnt = compute(1, cnt)
        start_out(a, 1)
        start_in(a + 2, 1)
        wait_in(0)
        wait_out(0)
        cnt = compute(0, cnt)
        start_out(bb, 0)
        return cnt

    cnt = lax.fori_loop(0, (_NCHUNK - 2) // 2, pair, cnt)
    wait_in(1)
    wait_out(1)
    cnt = compute(1, cnt)
    start_out(_NCHUNK - 1, 1)
    wait_out(0)
    wait_out(1)

    tn = pl.ds(2 * _MAIN, _TAIL)
    pltpu.sync_copy(lt.at[pl.ds(0, _C), pl.ds(b, 1), tn], tlbuf)
    pltpu.sync_copy(dbt.at[pl.ds(0, 4), tn], tdbuf)
    tinc = jnp.zeros((16,), jnp.int32)
    for j in range(_TAIL // 16):
        tinc = tinc + _process(tlbuf, tdbuf, tbbuf, tsbuf, tcbuf, j)
    cnt = cnt + jnp.where(half == 1, tinc, 0)
    pltpu.sync_copy(tbbuf, bx.at[pl.ds(b, 1), pl.ds(0, 4), tn])
    pltpu.sync_copy(tsbuf, sc.at[pl.ds(b, 1), tn])
    pltpu.sync_copy(tcbuf, cl.at[pl.ds(b, 1), tn])

    cntbuf[...] = cnt
    pltpu.sync_copy(cntbuf, cnts.at[wid])


@jax.jit
def _run(lt, dbt):
    mesh = plsc.VectorSubcoreMesh(core_axis_name="c", subcore_axis_name="s")
    kern = functools.partial(
        pl.kernel,
        mesh=mesh,
        out_type=[
            jax.ShapeDtypeStruct((_B, 4, _N), jnp.float32),
            jax.ShapeDtypeStruct((_B, _N), jnp.float32),
            jax.ShapeDtypeStruct((_B, _N), jnp.int32),
            jax.ShapeDtypeStruct((32, 16), jnp.int32),
        ],
        scratch_types=[
            pltpu.VMEM((_C, 1, _CH), jnp.float32),
            pltpu.VMEM((_C, 1, _CH), jnp.float32),
            pltpu.VMEM((4, _CH), jnp.float32),
            pltpu.VMEM((4, _CH), jnp.float32),
            pltpu.VMEM((1, 4, _CH), jnp.float32),
            pltpu.VMEM((1, 4, _CH), jnp.float32),
            pltpu.VMEM((1, _CH), jnp.float32),
            pltpu.VMEM((1, _CH), jnp.float32),
            pltpu.VMEM((1, _CH), jnp.int32),
            pltpu.VMEM((1, _CH), jnp.int32),
            pltpu.VMEM((_C, 1, _TAIL), jnp.float32),
            pltpu.VMEM((4, _TAIL), jnp.float32),
            pltpu.VMEM((1, 4, _TAIL), jnp.float32),
            pltpu.VMEM((1, _TAIL), jnp.float32),
            pltpu.VMEM((1, _TAIL), jnp.int32),
            pltpu.VMEM((16,), jnp.int32),
            pltpu.SemaphoreType.DMA,
            pltpu.SemaphoreType.DMA,
            pltpu.SemaphoreType.DMA,
            pltpu.SemaphoreType.DMA,
        ],
        compiler_params=pltpu.CompilerParams(
            needs_layout_passes=False, use_tc_tiling_on_sc=True),
    )(_decode_body)
    return kern(lt, dbt)


def kernel(logits, default_boxes):
    lt = jnp.transpose(logits, (2, 0, 1))
    dbt = jnp.transpose(default_boxes, (1, 0))
    boxes, scores, classes, counts = _run(lt, dbt)
    detection_boxes = jnp.transpose(boxes, (0, 2, 1))
    detection_num = counts.reshape(_B, 32).sum(axis=-1).astype(jnp.int32)
    return detection_boxes, scores, classes, detection_num
